# baseline (speedup 1.0000x reference)
import functools

import jax
import jax.numpy as jnp
from jax import lax
from jax.experimental import pallas as pl
from jax.experimental.pallas import tpu as pltpu
from jax.experimental.pallas import tpu_sc as plsc

BATCH = 4096
LENGTH = 200
HIDDEN = 64

LANE = 128
GROUP = 4
NUM_CORES = 2
NUM_SUBCORES = 16
NW = NUM_CORES * NUM_SUBCORES

N = BATCH * LENGTH
NBLK = N // LANE
BLK_PER_W = NBLK // NW
STEPS = BLK_PER_W // GROUP


def _emb_body(table_hbm, idx_hbm, out_hbm, idx_v, rows_v, sem):
    wid = lax.axis_index("s") * NUM_CORES + lax.axis_index("c")
    base = wid * BLK_PER_W

    def step(g, carry):
        r0 = base + g * GROUP
        pltpu.sync_copy(idx_hbm.at[pl.ds(r0, GROUP)], idx_v)
        copies = []
        for j in range(GROUP):
            copies.append(
                pltpu.async_copy(table_hbm.at[idx_v.at[j]], rows_v.at[j], sem)
            )
        for cp in copies:
            cp.wait()
        pltpu.sync_copy(rows_v, out_hbm.at[pl.ds(r0, GROUP)])
        return carry

    lax.fori_loop(0, STEPS, step, 0)


@functools.partial(jax.jit, donate_argnums=())
def _emb(weight, idx):
    mesh = plsc.VectorSubcoreMesh(core_axis_name="c", subcore_axis_name="s")
    k = pl.kernel(
        _emb_body,
        out_type=jax.ShapeDtypeStruct((NBLK, LANE, HIDDEN), jnp.float32),
        mesh=mesh,
        scratch_types=[
            pltpu.VMEM((GROUP, LANE), jnp.int32),
            pltpu.VMEM((GROUP, LANE, HIDDEN), jnp.float32),
            pltpu.SemaphoreType.DMA,
        ],
        compiler_params=pltpu.CompilerParams(use_tc_tiling_on_sc=False),
    )
    return k(weight, idx)


def kernel(inputs, weight):
    idx = inputs.reshape(NBLK, LANE).astype(jnp.int32)
    out = _emb(weight, idx)
    return out.reshape(BATCH, LENGTH, HIDDEN)


# speedup vs baseline: 1.0435x; 1.0435x over previous
import functools

import jax
import jax.numpy as jnp
from jax import lax
from jax.experimental import pallas as pl
from jax.experimental.pallas import tpu as pltpu
from jax.experimental.pallas import tpu_sc as plsc

BATCH = 4096
LENGTH = 200
HIDDEN = 64

LANE = 128
GROUP = 4
NUM_CORES = 2
NUM_SUBCORES = 16
NW = NUM_CORES * NUM_SUBCORES

N = BATCH * LENGTH
NBLK = N // LANE
BLK_PER_W = NBLK // NW
STEPS = BLK_PER_W // GROUP
PAIR = STEPS // 2


def _emb_body(table_hbm, idx_hbm, out_hbm, idx_v, rows_a, rows_b, sem):
    wid = lax.axis_index("s") * NUM_CORES + lax.axis_index("c")
    base = wid * BLK_PER_W

    pltpu.sync_copy(idx_hbm.at[pl.ds(base, BLK_PER_W)], idx_v)

    def fire(step, buf):
        r0 = step * GROUP
        for j in range(GROUP):
            pltpu.async_copy(table_hbm.at[idx_v.at[r0 + j]], buf.at[j], sem)

    def drain(buf):
        for j in range(GROUP):
            pltpu.make_async_copy(table_hbm.at[idx_v.at[j]], buf.at[j], sem).wait()

    def write(step, buf):
        pltpu.sync_copy(buf, out_hbm.at[pl.ds(base + step * GROUP, GROUP)])

    fire(0, rows_a)

    def body(t, carry):
        s0 = 2 * t
        fire(s0 + 1, rows_b)
        drain(rows_a)
        write(s0, rows_a)
        fire(lax.min(s0 + 2, STEPS - 2), rows_a)
        drain(rows_b)
        write(s0 + 1, rows_b)
        return carry

    lax.fori_loop(0, PAIR, body, 0)
    drain(rows_a)


@jax.jit
def _emb(weight, idx):
    mesh = plsc.VectorSubcoreMesh(core_axis_name="c", subcore_axis_name="s")
    k = pl.kernel(
        _emb_body,
        out_type=jax.ShapeDtypeStruct((NBLK, LANE, HIDDEN), jnp.float32),
        mesh=mesh,
        scratch_types=[
            pltpu.VMEM((BLK_PER_W, LANE), jnp.int32),
            pltpu.VMEM((GROUP, LANE, HIDDEN), jnp.float32),
            pltpu.VMEM((GROUP, LANE, HIDDEN), jnp.float32),
            pltpu.SemaphoreType.DMA,
        ],
        compiler_params=pltpu.CompilerParams(use_tc_tiling_on_sc=False),
    )
    return k(weight, idx)


def kernel(inputs, weight):
    idx = inputs.reshape(NBLK, LANE).astype(jnp.int32)
    out = _emb(weight, idx)
    return out.reshape(BATCH, LENGTH, HIDDEN)
